# baseline (speedup 1.0000x reference)
import functools

import jax
import jax.numpy as jnp
from jax import lax
from jax.experimental import pallas as pl
from jax.experimental.pallas import tpu as pltpu
from jax.experimental.pallas import tpu_sc as plsc

B = 16384
D = 64
MARGIN = 1.0
NC = 2
NS = 16
NW = NC * NS
BPW = B // NW
CB = 128
NCH = BPW // CB
NG = CB // 16


def _tec_body(ph_h, pt_h, pr_h, nh_h, nt_h, nr_h, ent_h, rel_h, out_h,
              ph_i, pt_i, pr_i, nh_i, nt_i, nr_i,
              hp_v, tp_v, rp_v, hn_v, tn_v, rn_v, lv, sem):
    wid = lax.axis_index("s") * NC + lax.axis_index("c")

    pltpu.sync_copy(ph_h.at[wid], ph_i)
    pltpu.sync_copy(pt_h.at[wid], pt_i)
    pltpu.sync_copy(pr_h.at[wid], pr_i)
    pltpu.sync_copy(nh_h.at[wid], nh_i)
    pltpu.sync_copy(nt_h.at[wid], nt_i)
    pltpu.sync_copy(nr_h.at[wid], nr_i)

    iota = lax.iota(jnp.int32, 16)
    DU = 8

    def chunk_body(c, loss16):
        cps = [
            pltpu.async_copy(ent_h.at[ph_i.at[c]], hp_v, sem),
            pltpu.async_copy(ent_h.at[pt_i.at[c]], tp_v, sem),
            pltpu.async_copy(rel_h.at[pr_i.at[c]], rp_v, sem),
            pltpu.async_copy(ent_h.at[nh_i.at[c]], hn_v, sem),
            pltpu.async_copy(ent_h.at[nt_i.at[c]], tn_v, sem),
            pltpu.async_copy(rel_h.at[nr_i.at[c]], rn_v, sem),
        ]
        for cp in cps:
            cp.wait()

        def group_body(g, acc):
            tri = g * 16 + iota

            def depth_body(db, accs):
                acc_p, acc_n = accs
                for j in range(DU):
                    col = db * DU + j + jnp.zeros((16,), jnp.int32)
                    hp = plsc.load_gather(hp_v, [tri, col])
                    tp = plsc.load_gather(tp_v, [tri, col])
                    rp = plsc.load_gather(rp_v, [tri, col])
                    hn = plsc.load_gather(hn_v, [tri, col])
                    tn = plsc.load_gather(tn_v, [tri, col])
                    rn = plsc.load_gather(rn_v, [tri, col])
                    acc_p = acc_p + jnp.abs(hp + rp - tp)
                    acc_n = acc_n + jnp.abs(hn + rn - tn)
                return acc_p, acc_n

            zero16 = jnp.zeros((16,), jnp.float32)
            acc_p, acc_n = lax.fori_loop(0, D // DU, depth_body, (zero16, zero16))
            return acc + jnp.maximum(acc_p - acc_n + MARGIN, 0.0)

        return lax.fori_loop(0, NG, group_body, loss16)

    lv[...] = lax.fori_loop(0, NCH, chunk_body, jnp.zeros((16,), jnp.float32))
    pltpu.sync_copy(lv, out_h.at[wid])


@functools.partial(jax.jit, static_argnums=())
def _run(ph, pt, pr, nh, nt, nr, ent, rel):
    mesh = plsc.VectorSubcoreMesh(core_axis_name="c", subcore_axis_name="s")
    k = pl.kernel(
        _tec_body,
        mesh=mesh,
        compiler_params=pltpu.CompilerParams(
            needs_layout_passes=False, use_tc_tiling_on_sc=False
        ),
        out_type=jax.ShapeDtypeStruct((NW, 16), jnp.float32),
        scratch_types=[
            pltpu.VMEM((NCH, CB), jnp.int32),
            pltpu.VMEM((NCH, CB), jnp.int32),
            pltpu.VMEM((NCH, CB), jnp.int32),
            pltpu.VMEM((NCH, CB), jnp.int32),
            pltpu.VMEM((NCH, CB), jnp.int32),
            pltpu.VMEM((NCH, CB), jnp.int32),
            pltpu.VMEM((CB, D), jnp.float32),
            pltpu.VMEM((CB, D), jnp.float32),
            pltpu.VMEM((CB, D), jnp.float32),
            pltpu.VMEM((CB, D), jnp.float32),
            pltpu.VMEM((CB, D), jnp.float32),
            pltpu.VMEM((CB, D), jnp.float32),
            pltpu.VMEM((16,), jnp.float32),
            pltpu.SemaphoreType.DMA,
        ],
    )
    return k(ph, pt, pr, nh, nt, nr, ent, rel)


def kernel(pos_h, pos_t, pos_r, neg_h, neg_t, neg_r, ent_embeddings, rel_embeddings):
    shp = (NW, NCH, CB)
    ph = pos_h.reshape(shp).astype(jnp.int32)
    pt = pos_t.reshape(shp).astype(jnp.int32)
    pr = pos_r.reshape(shp).astype(jnp.int32)
    nh = neg_h.reshape(shp).astype(jnp.int32)
    nt = neg_t.reshape(shp).astype(jnp.int32)
    nr = neg_r.reshape(shp).astype(jnp.int32)
    out = _run(ph, pt, pr, nh, nt, nr, ent_embeddings, rel_embeddings)
    return jnp.sum(out)


# speedup vs baseline: 1.6278x; 1.6278x over previous
import functools

import jax
import jax.numpy as jnp
from jax import lax
from jax.experimental import pallas as pl
from jax.experimental.pallas import tpu as pltpu
from jax.experimental.pallas import tpu_sc as plsc

B = 16384
D = 64
MARGIN = 1.0
NC = 2
NS = 16
NW = NC * NS
BPW = B // NW
NG = BPW // 16


def _tec_body(ph_h, pt_h, pr_h, nh_h, nt_h, nr_h, ent_h, rel_h, out_h,
              ph_i, pt_i, pr_i, nh_i, nt_i, nr_i,
              ra0, ra1, ra2, ra3, ra4, ra5,
              rb0, rb1, rb2, rb3, rb4, rb5,
              ov, sem):
    wid = lax.axis_index("s") * NC + lax.axis_index("c")

    pltpu.sync_copy(ph_h.at[wid], ph_i)
    pltpu.sync_copy(pt_h.at[wid], pt_i)
    pltpu.sync_copy(pr_h.at[wid], pr_i)
    pltpu.sync_copy(nh_h.at[wid], nh_i)
    pltpu.sync_copy(nt_h.at[wid], nt_i)
    pltpu.sync_copy(nr_h.at[wid], nr_i)

    idx_refs = (ph_i, pt_i, pr_i, nh_i, nt_i, nr_i)
    tabs = (ent_h, ent_h, rel_h, ent_h, ent_h, rel_h)
    slots = ((ra0, ra1, ra2, ra3, ra4, ra5), (rb0, rb1, rb2, rb3, rb4, rb5))

    def fire(g, slot):
        r, c0 = g >> 2, (g & 3) * 16
        for j in range(6):
            iv = idx_refs[j][r, pl.ds(c0, 16)]
            for i in range(16):
                pltpu.async_copy(
                    tabs[j].at[iv[i]], slots[slot][j].at[i], sem
                )

    def drain(slot):
        for j in range(6):
            for i in range(16):
                pltpu.make_async_copy(
                    tabs[j].at[0], slots[slot][j].at[i], sem
                ).wait()

    def score16(slot):
        bufs = slots[slot]
        total = 0.0
        for i in range(16):
            acc = jnp.zeros((16,), jnp.float32)
            for k in range(4):
                sl = pl.ds(k * 16, 16)
                hp = bufs[0][i, sl]
                tp = bufs[1][i, sl]
                rp = bufs[2][i, sl]
                hn = bufs[3][i, sl]
                tn = bufs[4][i, sl]
                rn = bufs[5][i, sl]
                acc = acc + (jnp.abs(hp + rp - tp) - jnp.abs(hn + rn - tn))
            total = total + jnp.maximum(jnp.sum(acc) + MARGIN, 0.0)
        return total

    fire(0, 0)

    def body(p, loss):
        g = p * 2
        fire(g + 1, 1)
        drain(0)
        loss = loss + score16(0)
        pl.when(g + 2 < NG)(lambda: fire(g + 2, 0))
        drain(1)
        return loss + score16(1)

    loss = lax.fori_loop(0, NG // 2, body, 0.0)

    z16 = jnp.zeros((16,), jnp.float32)
    for r in range(8):
        for kk in range(4):
            ov[r, pl.ds(kk * 16, 16)] = z16
    ov[0, pl.ds(0, 16)] = jnp.full((16,), loss * 0.0625, jnp.float32)
    pltpu.sync_copy(ov, out_h.at[wid])


@functools.partial(jax.jit, static_argnums=())
def _run(ph, pt, pr, nh, nt, nr, ent, rel):
    mesh = plsc.VectorSubcoreMesh(core_axis_name="c", subcore_axis_name="s")
    k = pl.kernel(
        _tec_body,
        mesh=mesh,
        compiler_params=pltpu.CompilerParams(
            needs_layout_passes=False, use_tc_tiling_on_sc=True
        ),
        out_type=jax.ShapeDtypeStruct((NW, 8, 64), jnp.float32),
        scratch_types=(
            [pltpu.VMEM((8, 64), jnp.int32) for _ in range(6)]
            + [pltpu.VMEM((16, 64), jnp.float32) for _ in range(12)]
            + [pltpu.VMEM((8, 64), jnp.float32), pltpu.SemaphoreType.DMA]
        ),
    )
    return k(ph, pt, pr, nh, nt, nr, ent, rel)


def kernel(pos_h, pos_t, pos_r, neg_h, neg_t, neg_r, ent_embeddings, rel_embeddings):
    shp = (NW, 8, 64)
    ph = pos_h.reshape(shp).astype(jnp.int32)
    pt = pos_t.reshape(shp).astype(jnp.int32)
    pr = pos_r.reshape(shp).astype(jnp.int32)
    nh = neg_h.reshape(shp).astype(jnp.int32)
    nt = neg_t.reshape(shp).astype(jnp.int32)
    nr = neg_r.reshape(shp).astype(jnp.int32)
    out = _run(ph, pt, pr, nh, nt, nr, ent_embeddings, rel_embeddings)
    return jnp.sum(out)


# speedup vs baseline: 1.6996x; 1.0441x over previous
import functools

import jax
import jax.numpy as jnp
from jax import lax
from jax.experimental import pallas as pl
from jax.experimental.pallas import tpu as pltpu
from jax.experimental.pallas import tpu_sc as plsc

B = 16384
D = 64
MARGIN = 1.0
NC = 2
NS = 16
NW = NC * NS
BPW = B // NW
NG = BPW // 16


def _tec_body(ph_h, pt_h, pr_h, nh_h, nt_h, nr_h, ent_h, rel_h, out_h,
              ph_i, pt_i, pr_i, nh_i, nt_i, nr_i,
              ra0, ra1, ra2, ra3, ra4, ra5,
              rb0, rb1, rb2, rb3, rb4, rb5,
              ov, sem):
    wid = lax.axis_index("s") * NC + lax.axis_index("c")

    pltpu.sync_copy(ph_h.at[wid], ph_i)
    pltpu.sync_copy(pt_h.at[wid], pt_i)
    pltpu.sync_copy(pr_h.at[wid], pr_i)
    pltpu.sync_copy(nh_h.at[wid], nh_i)
    pltpu.sync_copy(nt_h.at[wid], nt_i)
    pltpu.sync_copy(nr_h.at[wid], nr_i)

    idx_refs = (ph_i, pt_i, pr_i, nh_i, nt_i, nr_i)
    tabs = (ent_h, ent_h, rel_h, ent_h, ent_h, rel_h)
    slots = ((ra0, ra1, ra2, ra3, ra4, ra5), (rb0, rb1, rb2, rb3, rb4, rb5))

    def fire(g, slot):
        r, c0 = g >> 2, (g & 3) * 16
        for j in range(6):
            iv = idx_refs[j][r, pl.ds(c0, 16)]
            for i in range(16):
                pltpu.async_copy(
                    tabs[j].at[iv[i]], slots[slot][j].at[i], sem
                )

    def drain(slot):
        for j in range(6):
            pltpu.make_async_copy(
                tabs[j].at[pl.ds(0, 16)], slots[slot][j], sem
            ).wait()

    def score16(slot):
        bufs = slots[slot]
        total = 0.0
        for i in range(16):
            acc = jnp.zeros((16,), jnp.float32)
            for k in range(4):
                sl = pl.ds(k * 16, 16)
                hp = bufs[0][i, sl]
                tp = bufs[1][i, sl]
                rp = bufs[2][i, sl]
                hn = bufs[3][i, sl]
                tn = bufs[4][i, sl]
                rn = bufs[5][i, sl]
                acc = acc + (jnp.abs(hp + rp - tp) - jnp.abs(hn + rn - tn))
            total = total + jnp.maximum(jnp.sum(acc) + MARGIN, 0.0)
        return total

    fire(0, 0)

    def body(p, loss):
        g = p * 2
        fire(g + 1, 1)
        drain(0)
        loss = loss + score16(0)
        pl.when(g + 2 < NG)(lambda: fire(g + 2, 0))
        drain(1)
        return loss + score16(1)

    loss = lax.fori_loop(0, NG // 2, body, 0.0)

    z16 = jnp.zeros((16,), jnp.float32)
    for r in range(8):
        for kk in range(4):
            ov[r, pl.ds(kk * 16, 16)] = z16
    ov[0, pl.ds(0, 16)] = jnp.full((16,), loss * 0.0625, jnp.float32)
    pltpu.sync_copy(ov, out_h.at[wid])


@functools.partial(jax.jit, static_argnums=())
def _run(ph, pt, pr, nh, nt, nr, ent, rel):
    mesh = plsc.VectorSubcoreMesh(core_axis_name="c", subcore_axis_name="s")
    k = pl.kernel(
        _tec_body,
        mesh=mesh,
        compiler_params=pltpu.CompilerParams(
            needs_layout_passes=False, use_tc_tiling_on_sc=True
        ),
        out_type=jax.ShapeDtypeStruct((NW, 8, 64), jnp.float32),
        scratch_types=(
            [pltpu.VMEM((8, 64), jnp.int32) for _ in range(6)]
            + [pltpu.VMEM((16, 64), jnp.float32) for _ in range(12)]
            + [pltpu.VMEM((8, 64), jnp.float32), pltpu.SemaphoreType.DMA]
        ),
    )
    return k(ph, pt, pr, nh, nt, nr, ent, rel)


def kernel(pos_h, pos_t, pos_r, neg_h, neg_t, neg_r, ent_embeddings, rel_embeddings):
    shp = (NW, 8, 64)
    ph = pos_h.reshape(shp).astype(jnp.int32)
    pt = pos_t.reshape(shp).astype(jnp.int32)
    pr = pos_r.reshape(shp).astype(jnp.int32)
    nh = neg_h.reshape(shp).astype(jnp.int32)
    nt = neg_t.reshape(shp).astype(jnp.int32)
    nr = neg_r.reshape(shp).astype(jnp.int32)
    out = _run(ph, pt, pr, nh, nt, nr, ent_embeddings, rel_embeddings)
    return jnp.sum(out)
